# baseline (speedup 1.0000x reference)
import functools

import jax
import jax.numpy as jnp
from jax import lax
from jax.experimental import pallas as pl
from jax.experimental.pallas import tpu as pltpu
from jax.experimental.pallas import tpu_sc as plsc

N_NODES = 10000
N_EDGES = 320000
IN_CH = 128
HID = 64

NC = 2
NS = 16
NW = NC * NS
LANES = 16

N_PAD = 10240

GCHUNK = 128
NCHUNKS = N_EDGES // GCHUNK

CPW = HID // NW
SCHUNK = 2560
NSCHUNKS = N_EDGES // SCHUNK


def _node_mm_body(x_ref, w_ref, b_ref, out_ref):
    r = jnp.dot(x_ref[...], w_ref[...], preferred_element_type=jnp.float32)
    out_ref[...] = r + b_ref[...]


def _node_mm(x, wcat, bcat):
    n, k = x.shape
    blk = 2000
    return pl.pallas_call(
        _node_mm_body,
        grid=(n // blk,),
        in_specs=[
            pl.BlockSpec((blk, k), lambda i: (i, 0)),
            pl.BlockSpec((k, 2 * HID), lambda i: (0, 0)),
            pl.BlockSpec((1, 2 * HID), lambda i: (0, 0)),
        ],
        out_specs=pl.BlockSpec((blk, 2 * HID), lambda i: (i, 0)),
        out_shape=jax.ShapeDtypeStruct((n, 2 * HID), jnp.float32),
    )(x, wcat, bcat)


def _edge_mm_body(pre_ref, w_ref, b_ref, out_ref):
    a = jnp.maximum(pre_ref[...], 0.0)
    r = lax.dot_general(w_ref[...], a, (((0,), (0,)), ((), ())),
                        preferred_element_type=jnp.float32)
    out_ref[...] = r + b_ref[...]


def _edge_mm(pre_t, w, bcol):
    e = pre_t.shape[1]
    blk = 6400
    return pl.pallas_call(
        _edge_mm_body,
        grid=(e // blk,),
        in_specs=[
            pl.BlockSpec((HID, blk), lambda i: (0, i)),
            pl.BlockSpec((HID, HID), lambda i: (0, 0)),
            pl.BlockSpec((HID, 1), lambda i: (0, 0)),
        ],
        out_specs=pl.BlockSpec((HID, blk), lambda i: (0, i)),
        out_shape=jax.ShapeDtypeStruct((HID, e), jnp.float32),
    )(pre_t, w, bcol)


def _t_mm_body(lhs_ref, w_ref, b_ref, out_ref):
    r = lax.dot_general(lhs_ref[...], w_ref[...], (((0,), (0,)), ((), ())),
                        preferred_element_type=jnp.float32)
    out_ref[...] = r + b_ref[...]


def _t_mm(lhs_t, wcat, bcat):
    n = lhs_t.shape[1]
    return pl.pallas_call(
        _t_mm_body,
        grid=(1,),
        in_specs=[
            pl.BlockSpec((HID, n), lambda i: (0, 0)),
            pl.BlockSpec((HID, 2 * HID), lambda i: (0, 0)),
            pl.BlockSpec((1, 2 * HID), lambda i: (0, 0)),
        ],
        out_specs=pl.BlockSpec((n, 2 * HID), lambda i: (0, 0)),
        out_shape=jax.ShapeDtypeStruct((n, 2 * HID), jnp.float32),
    )(lhs_t, wcat, bcat)


def _head_mm_body(wt_ref, lhs_ref, b_ref, out_ref):
    r = jnp.dot(wt_ref[...], lhs_ref[...], preferred_element_type=jnp.float32)
    out_ref[...] = r + b_ref[...]


def _head_mm(lhs_t, wl_t, bl):
    n = lhs_t.shape[1]
    return pl.pallas_call(
        _head_mm_body,
        grid=(1,),
        in_specs=[
            pl.BlockSpec((1, HID), lambda i: (0, 0)),
            pl.BlockSpec((HID, n), lambda i: (0, 0)),
            pl.BlockSpec((1, 1), lambda i: (0, 0)),
        ],
        out_specs=pl.BlockSpec((1, n), lambda i: (0, 0)),
        out_shape=jax.ShapeDtypeStruct((1, n), jnp.float32),
    )(wl_t, lhs_t, bl)


def _sc_mesh():
    return plsc.VectorSubcoreMesh(
        core_axis_name="c", subcore_axis_name="s",
        num_cores=NC, num_subcores=NS)


def _edge_gather_body(c_hbm, dst_hbm, src_hbm, out_hbm,
                      idxd_v, idxs_v, bufd_v, bufs_v, outb_v, semd, sems):
    wid = lax.axis_index("s") * NC + lax.axis_index("c")
    nch = NCHUNKS // NW + jnp.where(wid < NCHUNKS % NW, 1, 0)

    lane = lax.iota(jnp.int32, LANES)

    def chunk(i, _):
        base = (wid + i * NW) * GCHUNK
        pltpu.sync_copy(dst_hbm.at[pl.ds(base, GCHUNK)], idxd_v)
        pltpu.sync_copy(src_hbm.at[pl.ds(base, GCHUNK)], idxs_v)
        cpa = pltpu.async_copy(c_hbm.at[idxd_v], bufd_v, semd)
        cpb = pltpu.async_copy(c_hbm.at[idxs_v], bufs_v, sems)
        cpa.wait()
        cpb.wait()

        def row(r, _):
            rcol = jnp.full((LANES,), r, jnp.int32)
            for s in range(HID // LANES):
                sl = pl.ds(s * LANES, LANES)
                sh = pl.ds(HID + s * LANES, LANES)
                v = bufd_v[r, sl] + bufs_v[r, sh]
                plsc.store_scatter(outb_v, [lane + s * LANES, rcol], v)
            return 0

        lax.fori_loop(0, GCHUNK, row, 0, unroll=2)
        pltpu.sync_copy(outb_v, out_hbm.at[:, pl.ds(base, GCHUNK)])
        return 0

    lax.fori_loop(0, nch, chunk, 0)


def _edge_gather(c, dst, src):
    kern = pl.kernel(
        _edge_gather_body,
        out_type=jax.ShapeDtypeStruct((HID, N_EDGES), jnp.float32),
        mesh=_sc_mesh(),
        compiler_params=pltpu.CompilerParams(needs_layout_passes=False),
        scratch_types=[
            pltpu.VMEM((GCHUNK,), jnp.int32),
            pltpu.VMEM((GCHUNK,), jnp.int32),
            pltpu.VMEM((GCHUNK, 2 * HID), jnp.float32),
            pltpu.VMEM((GCHUNK, 2 * HID), jnp.float32),
            pltpu.VMEM((HID, GCHUNK), jnp.float32),
            pltpu.SemaphoreType.DMA,
            pltpu.SemaphoreType.DMA,
        ],
    )
    return kern(c, dst, src)


def _seg_max_body(ht_hbm, dst_hbm, out_hbm, acc_v, dstb_v, hb_v):
    wid = lax.axis_index("s") * NC + lax.axis_index("c")
    c0 = wid * CPW

    def zero(i, _):
        for c in range(CPW):
            acc_v[c, pl.ds(i * LANES, LANES)] = jnp.zeros((LANES,), jnp.float32)
        return 0

    lax.fori_loop(0, N_PAD // LANES, zero, 0)

    def chunk(i, _):
        base = i * SCHUNK
        pltpu.sync_copy(dst_hbm.at[pl.ds(base, SCHUNK)], dstb_v)
        pltpu.sync_copy(ht_hbm.at[pl.ds(c0, CPW), pl.ds(base, SCHUNK)], hb_v)

        def vec(v, _):
            dv = dstb_v[pl.ds(v * LANES, LANES)]
            for c in range(CPW):
                cidx = jnp.full((LANES,), c, jnp.int32)
                h = hb_v[c, pl.ds(v * LANES, LANES)]
                cur = plsc.load_gather(acc_v, [cidx, dv])
                act = h > cur

                def cond(a):
                    return jnp.any(a)

                def body(a):
                    plsc.store_scatter(acc_v, [cidx, dv], h, mask=a)
                    got = plsc.load_gather(acc_v, [cidx, dv])
                    return a & (h > got)

                lax.while_loop(cond, body, act)
            return 0

        lax.fori_loop(0, SCHUNK // LANES, vec, 0)
        return 0

    lax.fori_loop(0, NSCHUNKS, chunk, 0)
    pltpu.sync_copy(acc_v, out_hbm.at[pl.ds(c0, CPW)])


def _seg_max(ht, dst):
    kern = pl.kernel(
        _seg_max_body,
        out_type=jax.ShapeDtypeStruct((HID, N_PAD), jnp.float32),
        mesh=_sc_mesh(),
        compiler_params=pltpu.CompilerParams(needs_layout_passes=False),
        scratch_types=[
            pltpu.VMEM((CPW, N_PAD), jnp.float32),
            pltpu.VMEM((SCHUNK,), jnp.int32),
            pltpu.VMEM((CPW, SCHUNK), jnp.float32),
        ],
    )
    return kern(ht, dst)


def kernel(x, edge_index, W1, b1, W2, b2, W3, b3, W4, b4, Wl, bl):
    src = edge_index[0].astype(jnp.int32)
    dst = edge_index[1].astype(jnp.int32)

    w1cat = jnp.concatenate([W1[:IN_CH] - W1[IN_CH:], W1[IN_CH:]], axis=1)
    b1cat = jnp.concatenate([b1, jnp.zeros_like(b1)])[None, :]
    c1 = _node_mm(x, w1cat, b1cat)
    pre1 = _edge_gather(c1, dst, src)
    h1t = _seg_max(_edge_mm(pre1, W2, b2[:, None]), dst)

    w3cat = jnp.concatenate([W3[:HID] - W3[HID:], W3[HID:]], axis=1)
    b3cat = jnp.concatenate([b3, jnp.zeros_like(b3)])[None, :]
    c2 = _t_mm(h1t, w3cat, b3cat)
    pre2 = _edge_gather(c2, dst, src)
    h2t = _seg_max(_edge_mm(pre2, W4, b4[:, None]), dst)

    out = _head_mm(h2t, Wl.T, bl[None, :])
    return out[0, :N_NODES]


# speedup vs baseline: 1.7780x; 1.7780x over previous
import functools

import jax
import jax.numpy as jnp
from jax import lax
from jax.experimental import pallas as pl
from jax.experimental.pallas import tpu as pltpu
from jax.experimental.pallas import tpu_sc as plsc

N_NODES = 10000
N_EDGES = 320000
IN_CH = 128
HID = 64

NC = 2
NS = 16
NW = NC * NS
LANES = 16

N_PAD = 10240

GCHUNK = 128
NCHUNKS = N_EDGES // GCHUNK

CPW = HID // NW
SCHUNK = 2560
NSCHUNKS = N_EDGES // SCHUNK


def _node_mm_body(x_ref, w_ref, b_ref, out_ref):
    r = jnp.dot(x_ref[...], w_ref[...], preferred_element_type=jnp.float32)
    out_ref[...] = r + b_ref[...]


def _node_mm(x, wcat, bcat):
    n, k = x.shape
    blk = 2000
    return pl.pallas_call(
        _node_mm_body,
        grid=(n // blk,),
        in_specs=[
            pl.BlockSpec((blk, k), lambda i: (i, 0)),
            pl.BlockSpec((k, 2 * HID), lambda i: (0, 0)),
            pl.BlockSpec((1, 2 * HID), lambda i: (0, 0)),
        ],
        out_specs=pl.BlockSpec((blk, 2 * HID), lambda i: (i, 0)),
        out_shape=jax.ShapeDtypeStruct((n, 2 * HID), jnp.float32),
    )(x, wcat, bcat)


def _edge_mm_body(pre_ref, w_ref, b_ref, out_ref):
    a = jnp.maximum(pre_ref[...], 0.0)
    r = lax.dot_general(w_ref[...], a, (((0,), (0,)), ((), ())),
                        preferred_element_type=jnp.float32)
    out_ref[...] = r + b_ref[...]


def _edge_mm(pre_t, w, bcol):
    e = pre_t.shape[1]
    blk = 6400
    return pl.pallas_call(
        _edge_mm_body,
        grid=(e // blk,),
        in_specs=[
            pl.BlockSpec((HID, blk), lambda i: (0, i)),
            pl.BlockSpec((HID, HID), lambda i: (0, 0)),
            pl.BlockSpec((HID, 1), lambda i: (0, 0)),
        ],
        out_specs=pl.BlockSpec((HID, blk), lambda i: (0, i)),
        out_shape=jax.ShapeDtypeStruct((HID, e), jnp.float32),
    )(pre_t, w, bcol)


def _t_mm_body(lhs_ref, w_ref, b_ref, out_ref):
    r = lax.dot_general(lhs_ref[...], w_ref[...], (((0,), (0,)), ((), ())),
                        preferred_element_type=jnp.float32)
    out_ref[...] = r + b_ref[...]


def _t_mm(lhs_t, wcat, bcat):
    n = lhs_t.shape[1]
    return pl.pallas_call(
        _t_mm_body,
        grid=(1,),
        in_specs=[
            pl.BlockSpec((HID, n), lambda i: (0, 0)),
            pl.BlockSpec((HID, 2 * HID), lambda i: (0, 0)),
            pl.BlockSpec((1, 2 * HID), lambda i: (0, 0)),
        ],
        out_specs=pl.BlockSpec((n, 2 * HID), lambda i: (0, 0)),
        out_shape=jax.ShapeDtypeStruct((n, 2 * HID), jnp.float32),
    )(lhs_t, wcat, bcat)


def _head_mm_body(wt_ref, lhs_ref, b_ref, out_ref):
    r = jnp.dot(wt_ref[...], lhs_ref[...], preferred_element_type=jnp.float32)
    out_ref[...] = r + b_ref[...]


def _head_mm(lhs_t, wl_t, bl):
    n = lhs_t.shape[1]
    return pl.pallas_call(
        _head_mm_body,
        grid=(1,),
        in_specs=[
            pl.BlockSpec((1, HID), lambda i: (0, 0)),
            pl.BlockSpec((HID, n), lambda i: (0, 0)),
            pl.BlockSpec((1, 1), lambda i: (0, 0)),
        ],
        out_specs=pl.BlockSpec((1, n), lambda i: (0, 0)),
        out_shape=jax.ShapeDtypeStruct((1, n), jnp.float32),
    )(wl_t, lhs_t, bl)


def _sc_mesh():
    return plsc.VectorSubcoreMesh(
        core_axis_name="c", subcore_axis_name="s",
        num_cores=NC, num_subcores=NS)


def _edge_gather_body(c_hbm, dst_hbm, src_hbm, out_hbm,
                      idxd_v, idxs_v, bufd_v, bufs_v, outb_v, semd, sems):
    wid = lax.axis_index("s") * NC + lax.axis_index("c")
    nch = NCHUNKS // NW + jnp.where(wid < NCHUNKS % NW, 1, 0)

    lane = lax.iota(jnp.int32, LANES)

    def chunk(i, _):
        base = (wid + i * NW) * GCHUNK
        pltpu.sync_copy(dst_hbm.at[pl.ds(base, GCHUNK)], idxd_v)
        pltpu.sync_copy(src_hbm.at[pl.ds(base, GCHUNK)], idxs_v)
        cpa = pltpu.async_copy(c_hbm.at[idxd_v], bufd_v, semd)
        cpb = pltpu.async_copy(c_hbm.at[idxs_v], bufs_v, sems)
        cpa.wait()
        cpb.wait()

        def row(r, _):
            rcol = jnp.full((LANES,), r, jnp.int32)
            for s in range(HID // LANES):
                sl = pl.ds(s * LANES, LANES)
                sh = pl.ds(HID + s * LANES, LANES)
                v = bufd_v[r, sl] + bufs_v[r, sh]
                plsc.store_scatter(outb_v, [lane + s * LANES, rcol], v)
            return 0

        lax.fori_loop(0, GCHUNK, row, 0, unroll=2)
        pltpu.sync_copy(outb_v, out_hbm.at[:, pl.ds(base, GCHUNK)])
        return 0

    lax.fori_loop(0, nch, chunk, 0)


def _edge_gather(c, dst, src):
    kern = pl.kernel(
        _edge_gather_body,
        out_type=jax.ShapeDtypeStruct((HID, N_EDGES), jnp.float32),
        mesh=_sc_mesh(),
        compiler_params=pltpu.CompilerParams(needs_layout_passes=False),
        scratch_types=[
            pltpu.VMEM((GCHUNK,), jnp.int32),
            pltpu.VMEM((GCHUNK,), jnp.int32),
            pltpu.VMEM((GCHUNK, 2 * HID), jnp.float32),
            pltpu.VMEM((GCHUNK, 2 * HID), jnp.float32),
            pltpu.VMEM((HID, GCHUNK), jnp.float32),
            pltpu.SemaphoreType.DMA,
            pltpu.SemaphoreType.DMA,
        ],
    )
    return kern(c, dst, src)


_SPILL_CAP = (SCHUNK // LANES) * CPW * LANES


def _seg_max_body(ht_hbm, dst_hbm, out_hbm,
                  acc_v, dstb_v, hb_v, spd_v, spv_v):
    wid = lax.axis_index("s") * NC + lax.axis_index("c")
    c0 = wid * CPW
    lane = lax.iota(jnp.int32, LANES)

    def zero(i, _):
        acc_v[pl.ds(i * LANES, LANES)] = jnp.zeros((LANES,), jnp.float32)
        return 0

    lax.fori_loop(0, CPW * N_PAD // LANES, zero, 0)

    def chunk(i, _):
        base = i * SCHUNK
        pltpu.sync_copy(dst_hbm.at[pl.ds(base, SCHUNK)], dstb_v)
        pltpu.sync_copy(ht_hbm.at[pl.ds(c0, CPW), pl.ds(base, SCHUNK)], hb_v)

        def vec(v, off):
            dv = dstb_v[pl.ds(v * LANES, LANES)]
            for c in range(CPW):
                idx = dv + c * N_PAD if c else dv
                h = hb_v[c, pl.ds(v * LANES, LANES)]
                got = plsc.load_gather(acc_v, [idx])
                m = jnp.maximum(h, got)
                plsc.store_scatter(acc_v, [idx], m)
                got2 = plsc.load_gather(acc_v, [idx])
                lost = m > got2
                pcnt = plsc.all_reduce_population_count(lost)
                plsc.store_scatter(spd_v, [off + lane],
                                   jnp.where(lost, idx, 0))
                plsc.store_scatter(spv_v, [off + lane],
                                   jnp.where(lost, m, 0.0))
                off = off + jnp.where(pcnt > 0, LANES, 0)
            return off

        off = lax.fori_loop(0, SCHUNK // LANES, vec,
                            jnp.zeros((LANES,), jnp.int32))
        n = off[0]

        def replay(j, _):
            sdv = spd_v[pl.ds(j * LANES, LANES)]
            sv = spv_v[pl.ds(j * LANES, LANES)]
            got = plsc.load_gather(acc_v, [sdv])
            act = sv > got

            def cond(a):
                return jnp.any(a)

            def body(a):
                plsc.store_scatter(acc_v, [sdv], sv, mask=a)
                g = plsc.load_gather(acc_v, [sdv])
                return a & (sv > g)

            lax.while_loop(cond, body, act)
            return 0

        lax.fori_loop(0, n // LANES, replay, 0)
        return 0

    lax.fori_loop(0, NSCHUNKS, chunk, 0)
    for c in range(CPW):
        pltpu.sync_copy(acc_v.at[pl.ds(c * N_PAD, N_PAD)], out_hbm.at[c0 + c])


def _seg_max(ht, dst):
    kern = pl.kernel(
        _seg_max_body,
        out_type=jax.ShapeDtypeStruct((HID, N_PAD), jnp.float32),
        mesh=_sc_mesh(),
        compiler_params=pltpu.CompilerParams(needs_layout_passes=False),
        scratch_types=[
            pltpu.VMEM((CPW * N_PAD,), jnp.float32),
            pltpu.VMEM((SCHUNK,), jnp.int32),
            pltpu.VMEM((CPW, SCHUNK), jnp.float32),
            pltpu.VMEM((_SPILL_CAP,), jnp.int32),
            pltpu.VMEM((_SPILL_CAP,), jnp.float32),
        ],
    )
    return kern(ht, dst)


def kernel(x, edge_index, W1, b1, W2, b2, W3, b3, W4, b4, Wl, bl):
    src = edge_index[0].astype(jnp.int32)
    dst = edge_index[1].astype(jnp.int32)

    w1cat = jnp.concatenate([W1[:IN_CH] - W1[IN_CH:], W1[IN_CH:]], axis=1)
    b1cat = jnp.concatenate([b1, jnp.zeros_like(b1)])[None, :]
    c1 = _node_mm(x, w1cat, b1cat)
    pre1 = _edge_gather(c1, dst, src)
    h1t = _seg_max(_edge_mm(pre1, W2, b2[:, None]), dst)

    w3cat = jnp.concatenate([W3[:HID] - W3[HID:], W3[HID:]], axis=1)
    b3cat = jnp.concatenate([b3, jnp.zeros_like(b3)])[None, :]
    c2 = _t_mm(h1t, w3cat, b3cat)
    pre2 = _edge_gather(c2, dst, src)
    h2t = _seg_max(_edge_mm(pre2, W4, b4[:, None]), dst)

    out = _head_mm(h2t, Wl.T, bl[None, :])
    return out[0, :N_NODES]


# speedup vs baseline: 1.9605x; 1.1027x over previous
import functools

import jax
import jax.numpy as jnp
from jax import lax
from jax.experimental import pallas as pl
from jax.experimental.pallas import tpu as pltpu
from jax.experimental.pallas import tpu_sc as plsc

N_NODES = 10000
N_EDGES = 320000
IN_CH = 128
HID = 64

NC = 2
NS = 16
NW = NC * NS
LANES = 16

N_PAD = 10240

GCHUNK = 128
NCHUNKS = N_EDGES // GCHUNK

CPW = HID // NW
SCHUNK = 2560
NSCHUNKS = N_EDGES // SCHUNK


def _node_mm_body(x_ref, w_ref, b_ref, out_ref):
    r = jnp.dot(x_ref[...], w_ref[...], preferred_element_type=jnp.float32, precision=lax.Precision.HIGHEST)
    out_ref[...] = r + b_ref[...]


def _node_mm(x, wcat, bcat):
    n, k = x.shape
    blk = 2000
    return pl.pallas_call(
        _node_mm_body,
        grid=(n // blk,),
        in_specs=[
            pl.BlockSpec((blk, k), lambda i: (i, 0)),
            pl.BlockSpec((k, 2 * HID), lambda i: (0, 0)),
            pl.BlockSpec((1, 2 * HID), lambda i: (0, 0)),
        ],
        out_specs=pl.BlockSpec((blk, 2 * HID), lambda i: (i, 0)),
        out_shape=jax.ShapeDtypeStruct((n, 2 * HID), jnp.float32),
    )(x, wcat, bcat)


def _edge_mm_body(pre_ref, w_ref, b_ref, out_ref):
    a = jnp.maximum(pre_ref[...], 0.0)
    r = lax.dot_general(w_ref[...], a, (((0,), (0,)), ((), ())),
                        preferred_element_type=jnp.float32,
                        precision=lax.Precision.HIGHEST)
    out_ref[...] = r + b_ref[...]


def _edge_mm(pre_t, w, bcol):
    e = pre_t.shape[1]
    blk = 6400
    return pl.pallas_call(
        _edge_mm_body,
        grid=(e // blk,),
        in_specs=[
            pl.BlockSpec((HID, blk), lambda i: (0, i)),
            pl.BlockSpec((HID, HID), lambda i: (0, 0)),
            pl.BlockSpec((HID, 1), lambda i: (0, 0)),
        ],
        out_specs=pl.BlockSpec((HID, blk), lambda i: (0, i)),
        out_shape=jax.ShapeDtypeStruct((HID, e), jnp.float32),
    )(pre_t, w, bcol)


def _t_mm_body(lhs_ref, w_ref, b_ref, out_ref):
    r = lax.dot_general(lhs_ref[...], w_ref[...], (((0,), (0,)), ((), ())),
                        preferred_element_type=jnp.float32,
                        precision=lax.Precision.HIGHEST)
    out_ref[...] = r + b_ref[...]


def _t_mm(lhs_t, wcat, bcat):
    n = lhs_t.shape[1]
    return pl.pallas_call(
        _t_mm_body,
        grid=(1,),
        in_specs=[
            pl.BlockSpec((HID, n), lambda i: (0, 0)),
            pl.BlockSpec((HID, 2 * HID), lambda i: (0, 0)),
            pl.BlockSpec((1, 2 * HID), lambda i: (0, 0)),
        ],
        out_specs=pl.BlockSpec((n, 2 * HID), lambda i: (0, 0)),
        out_shape=jax.ShapeDtypeStruct((n, 2 * HID), jnp.float32),
    )(lhs_t, wcat, bcat)


def _head_mm_body(wt_ref, lhs_ref, b_ref, out_ref):
    r = jnp.dot(wt_ref[...], lhs_ref[...], preferred_element_type=jnp.float32, precision=lax.Precision.HIGHEST)
    out_ref[...] = r + b_ref[...]


def _head_mm(lhs_t, wl_t, bl):
    n = lhs_t.shape[1]
    return pl.pallas_call(
        _head_mm_body,
        grid=(1,),
        in_specs=[
            pl.BlockSpec((1, HID), lambda i: (0, 0)),
            pl.BlockSpec((HID, n), lambda i: (0, 0)),
            pl.BlockSpec((1, 1), lambda i: (0, 0)),
        ],
        out_specs=pl.BlockSpec((1, n), lambda i: (0, 0)),
        out_shape=jax.ShapeDtypeStruct((1, n), jnp.float32),
    )(wl_t, lhs_t, bl)


def _sc_mesh():
    return plsc.VectorSubcoreMesh(
        core_axis_name="c", subcore_axis_name="s",
        num_cores=NC, num_subcores=NS)


_GCH_UNIFORM = NCHUNKS // NW
_GCH_REM = NCHUNKS - _GCH_UNIFORM * NW


def _edge_gather_body(c_hbm, dst_hbm, src_hbm, out_hbm,
                      idxd_v, idxs_v, bufd_v, bufs_v, outb_v,
                      semd0, semd1, sems0, sems1, semw0, semw1):
    wid = lax.axis_index("s") * NC + lax.axis_index("c")
    lane = lax.iota(jnp.int32, LANES)
    semd = (semd0, semd1)
    sems = (sems0, sems1)
    semw = (semw0, semw1)

    def load_idx_and_gather(i, b):
        j = wid + i * NW
        pltpu.sync_copy(dst_hbm.at[j], idxd_v.at[b])
        pltpu.sync_copy(src_hbm.at[j], idxs_v.at[b])
        pltpu.async_copy(c_hbm.at[idxd_v.at[b]], bufd_v.at[b], semd[b])
        pltpu.async_copy(c_hbm.at[idxs_v.at[b]], bufs_v.at[b], sems[b])

    def wait_gather(b):
        pltpu.make_async_copy(c_hbm.at[idxd_v.at[b]], bufd_v.at[b], semd[b]).wait()
        pltpu.make_async_copy(c_hbm.at[idxs_v.at[b]], bufs_v.at[b], sems[b]).wait()

    def wait_wb(b):
        pltpu.make_async_copy(outb_v.at[b],
                              out_hbm.at[:, pl.ds(0, GCHUNK)], semw[b]).wait()

    def compute(b):
        def row(r, _):
            rcol = jnp.full((LANES,), r, jnp.int32)
            for s in range(HID // LANES):
                sl = pl.ds(s * LANES, LANES)
                sh = pl.ds(HID + s * LANES, LANES)
                v = bufd_v[b, r, sl] + bufs_v[b, r, sh]
                plsc.store_scatter(outb_v.at[b], [lane + s * LANES, rcol], v)
            return 0

        lax.fori_loop(0, GCHUNK, row, 0, unroll=2)

    load_idx_and_gather(0, 0)

    def group(g, _):
        for b in range(2):
            i = g * 2 + b

            @pl.when(i >= 2)
            def _():
                wait_wb(b)

            wait_gather(b)

            @pl.when(i + 1 < _GCH_UNIFORM)
            def _():
                load_idx_and_gather(i + 1, 1 - b)

            compute(b)
            base = (wid + i * NW) * GCHUNK
            pltpu.async_copy(outb_v.at[b],
                             out_hbm.at[:, pl.ds(base, GCHUNK)], semw[b])
        return 0

    lax.fori_loop(0, _GCH_UNIFORM // 2, group, 0)
    wait_wb(0)
    wait_wb(1)

    @pl.when(wid < _GCH_REM)
    def _():
        j = _GCH_UNIFORM * NW + wid
        pltpu.sync_copy(dst_hbm.at[j], idxd_v.at[0])
        pltpu.sync_copy(src_hbm.at[j], idxs_v.at[0])
        pltpu.async_copy(c_hbm.at[idxd_v.at[0]], bufd_v.at[0], semd[0])
        pltpu.async_copy(c_hbm.at[idxs_v.at[0]], bufs_v.at[0], sems[0])
        wait_gather(0)
        compute(0)
        pltpu.sync_copy(outb_v.at[0], out_hbm.at[:, pl.ds(j * GCHUNK, GCHUNK)])


def _edge_gather(c, dst2d, src2d):
    kern = pl.kernel(
        _edge_gather_body,
        out_type=jax.ShapeDtypeStruct((HID, N_EDGES), jnp.float32),
        mesh=_sc_mesh(),
        compiler_params=pltpu.CompilerParams(needs_layout_passes=False),
        scratch_types=[
            pltpu.VMEM((2, GCHUNK), jnp.int32),
            pltpu.VMEM((2, GCHUNK), jnp.int32),
            pltpu.VMEM((2, GCHUNK, 2 * HID), jnp.float32),
            pltpu.VMEM((2, GCHUNK, 2 * HID), jnp.float32),
            pltpu.VMEM((2, HID, GCHUNK), jnp.float32),
            pltpu.SemaphoreType.DMA,
            pltpu.SemaphoreType.DMA,
            pltpu.SemaphoreType.DMA,
            pltpu.SemaphoreType.DMA,
            pltpu.SemaphoreType.DMA,
            pltpu.SemaphoreType.DMA,
        ],
    )
    return kern(c, dst2d, src2d)


_SPILL_CAP = (SCHUNK // LANES) * CPW * LANES


def _seg_max_body(ht_hbm, dst_hbm, out_hbm,
                  acc_v, dstb_v, hb_v, spd_v, spv_v):
    wid = lax.axis_index("s") * NC + lax.axis_index("c")
    c0 = wid * CPW
    lane = lax.iota(jnp.int32, LANES)

    def zero(i, _):
        acc_v[pl.ds(i * LANES, LANES)] = jnp.zeros((LANES,), jnp.float32)
        return 0

    lax.fori_loop(0, CPW * N_PAD // LANES, zero, 0)

    def chunk(i, _):
        base = i * SCHUNK
        pltpu.sync_copy(dst_hbm.at[pl.ds(base, SCHUNK)], dstb_v)
        pltpu.sync_copy(ht_hbm.at[pl.ds(c0, CPW), pl.ds(base, SCHUNK)], hb_v)

        def vec(v, off):
            dv = dstb_v[pl.ds(v * LANES, LANES)]
            for c in range(CPW):
                idx = dv + c * N_PAD if c else dv
                h = hb_v[c, pl.ds(v * LANES, LANES)]
                got = plsc.load_gather(acc_v, [idx])
                m = jnp.maximum(h, got)
                plsc.store_scatter(acc_v, [idx], m)
                got2 = plsc.load_gather(acc_v, [idx])
                lost = m > got2
                pcnt = plsc.all_reduce_population_count(lost)
                plsc.store_scatter(spd_v, [off + lane],
                                   jnp.where(lost, idx, 0))
                plsc.store_scatter(spv_v, [off + lane],
                                   jnp.where(lost, m, 0.0))
                off = off + jnp.where(pcnt > 0, LANES, 0)
            return off

        off = lax.fori_loop(0, SCHUNK // LANES, vec,
                            jnp.zeros((LANES,), jnp.int32))
        n = off[0]

        def replay(j, _):
            sdv = spd_v[pl.ds(j * LANES, LANES)]
            sv = spv_v[pl.ds(j * LANES, LANES)]
            got = plsc.load_gather(acc_v, [sdv])
            act = sv > got

            def cond(a):
                return jnp.any(a)

            def body(a):
                plsc.store_scatter(acc_v, [sdv], sv, mask=a)
                g = plsc.load_gather(acc_v, [sdv])
                return a & (sv > g)

            lax.while_loop(cond, body, act)
            return 0

        lax.fori_loop(0, n // LANES, replay, 0)
        return 0

    lax.fori_loop(0, NSCHUNKS, chunk, 0)
    for c in range(CPW):
        pltpu.sync_copy(acc_v.at[pl.ds(c * N_PAD, N_PAD)], out_hbm.at[c0 + c])


def _seg_max(ht, dst):
    kern = pl.kernel(
        _seg_max_body,
        out_type=jax.ShapeDtypeStruct((HID, N_PAD), jnp.float32),
        mesh=_sc_mesh(),
        compiler_params=pltpu.CompilerParams(needs_layout_passes=False),
        scratch_types=[
            pltpu.VMEM((CPW * N_PAD,), jnp.float32),
            pltpu.VMEM((SCHUNK,), jnp.int32),
            pltpu.VMEM((CPW, SCHUNK), jnp.float32),
            pltpu.VMEM((_SPILL_CAP,), jnp.int32),
            pltpu.VMEM((_SPILL_CAP,), jnp.float32),
        ],
    )
    return kern(ht, dst)


def kernel(x, edge_index, W1, b1, W2, b2, W3, b3, W4, b4, Wl, bl):
    src = edge_index[0].astype(jnp.int32)
    dst = edge_index[1].astype(jnp.int32)
    src2d = src.reshape(NCHUNKS, GCHUNK)
    dst2d = dst.reshape(NCHUNKS, GCHUNK)

    w1cat = jnp.concatenate([W1[:IN_CH] - W1[IN_CH:], W1[IN_CH:]], axis=1)
    b1cat = jnp.concatenate([b1, jnp.zeros_like(b1)])[None, :]
    c1 = _node_mm(x, w1cat, b1cat)
    pre1 = _edge_gather(c1, dst2d, src2d)
    h1t = _seg_max(_edge_mm(pre1, W2, b2[:, None]), dst)

    w3cat = jnp.concatenate([W3[:HID] - W3[HID:], W3[HID:]], axis=1)
    b3cat = jnp.concatenate([b3, jnp.zeros_like(b3)])[None, :]
    c2 = _t_mm(h1t, w3cat, b3cat)
    pre2 = _edge_gather(c2, dst2d, src2d)
    h2t = _seg_max(_edge_mm(pre2, W4, b4[:, None]), dst)

    out = _head_mm(h2t, Wl.T, bl[None, :])
    return out[0, :N_NODES]


# speedup vs baseline: 2.0695x; 1.0556x over previous
import functools

import jax
import jax.numpy as jnp
from jax import lax
from jax.experimental import pallas as pl
from jax.experimental.pallas import tpu as pltpu
from jax.experimental.pallas import tpu_sc as plsc

N_NODES = 10000
N_EDGES = 320000
IN_CH = 128
HID = 64

NC = 2
NS = 16
NW = NC * NS
LANES = 16

N_PAD = 10240

GCHUNK = 128
NCHUNKS = N_EDGES // GCHUNK

CPW = HID // NW
SCHUNK = 2560
NSCHUNKS = N_EDGES // SCHUNK


def _node_mm_body(x_ref, w_ref, b_ref, out_ref):
    r = jnp.dot(x_ref[...], w_ref[...], preferred_element_type=jnp.float32, precision=lax.Precision.HIGHEST)
    out_ref[...] = r + b_ref[...]


def _node_mm(x, wcat, bcat):
    n, k = x.shape
    blk = 2000
    return pl.pallas_call(
        _node_mm_body,
        grid=(n // blk,),
        in_specs=[
            pl.BlockSpec((blk, k), lambda i: (i, 0)),
            pl.BlockSpec((k, 2 * HID), lambda i: (0, 0)),
            pl.BlockSpec((1, 2 * HID), lambda i: (0, 0)),
        ],
        out_specs=pl.BlockSpec((blk, 2 * HID), lambda i: (i, 0)),
        out_shape=jax.ShapeDtypeStruct((n, 2 * HID), jnp.float32),
    )(x, wcat, bcat)


def _edge_mm_body(pre_ref, w_ref, b_ref, out_ref):
    a = jnp.maximum(pre_ref[...], 0.0)
    r = lax.dot_general(w_ref[...], a, (((0,), (0,)), ((), ())),
                        preferred_element_type=jnp.float32,
                        precision=lax.Precision.HIGHEST)
    out_ref[...] = r + b_ref[...]


def _edge_mm(pre_t, w, bcol):
    e = pre_t.shape[1]
    blk = 6400
    return pl.pallas_call(
        _edge_mm_body,
        grid=(e // blk,),
        in_specs=[
            pl.BlockSpec((HID, blk), lambda i: (0, i)),
            pl.BlockSpec((HID, HID), lambda i: (0, 0)),
            pl.BlockSpec((HID, 1), lambda i: (0, 0)),
        ],
        out_specs=pl.BlockSpec((HID, blk), lambda i: (0, i)),
        out_shape=jax.ShapeDtypeStruct((HID, e), jnp.float32),
    )(pre_t, w, bcol)


def _t_mm_body(lhs_ref, w_ref, b_ref, out_ref):
    r = lax.dot_general(lhs_ref[...], w_ref[...], (((0,), (0,)), ((), ())),
                        preferred_element_type=jnp.float32,
                        precision=lax.Precision.HIGHEST)
    out_ref[...] = r + b_ref[...]


def _t_mm(lhs_t, wcat, bcat):
    n = lhs_t.shape[1]
    return pl.pallas_call(
        _t_mm_body,
        grid=(1,),
        in_specs=[
            pl.BlockSpec((HID, n), lambda i: (0, 0)),
            pl.BlockSpec((HID, 2 * HID), lambda i: (0, 0)),
            pl.BlockSpec((1, 2 * HID), lambda i: (0, 0)),
        ],
        out_specs=pl.BlockSpec((n, 2 * HID), lambda i: (0, 0)),
        out_shape=jax.ShapeDtypeStruct((n, 2 * HID), jnp.float32),
    )(lhs_t, wcat, bcat)


def _head_mm_body(wt_ref, lhs_ref, b_ref, out_ref):
    r = jnp.dot(wt_ref[...], lhs_ref[...], preferred_element_type=jnp.float32, precision=lax.Precision.HIGHEST)
    out_ref[...] = r + b_ref[...]


def _head_mm(lhs_t, wl_t, bl):
    n = lhs_t.shape[1]
    return pl.pallas_call(
        _head_mm_body,
        grid=(1,),
        in_specs=[
            pl.BlockSpec((1, HID), lambda i: (0, 0)),
            pl.BlockSpec((HID, n), lambda i: (0, 0)),
            pl.BlockSpec((1, 1), lambda i: (0, 0)),
        ],
        out_specs=pl.BlockSpec((1, n), lambda i: (0, 0)),
        out_shape=jax.ShapeDtypeStruct((1, n), jnp.float32),
    )(wl_t, lhs_t, bl)


def _sc_mesh():
    return plsc.VectorSubcoreMesh(
        core_axis_name="c", subcore_axis_name="s",
        num_cores=NC, num_subcores=NS)


_GCH_UNIFORM = NCHUNKS // NW
_GCH_REM = NCHUNKS - _GCH_UNIFORM * NW


def _edge_gather_body(c_hbm, dst_hbm, src_hbm, out_hbm,
                      idxd_v, idxs_v, bufd_v, bufs_v, outb_v,
                      semd0, semd1, sems0, sems1, semw0, semw1):
    wid = lax.axis_index("s") * NC + lax.axis_index("c")
    lane = lax.iota(jnp.int32, LANES)
    semd = (semd0, semd1)
    sems = (sems0, sems1)
    semw = (semw0, semw1)

    def load_idx_and_gather(i, b):
        j = wid + i * NW
        pltpu.sync_copy(dst_hbm.at[j], idxd_v.at[b])
        pltpu.sync_copy(src_hbm.at[j], idxs_v.at[b])
        pltpu.async_copy(c_hbm.at[idxd_v.at[b]], bufd_v.at[b], semd[b])
        pltpu.async_copy(c_hbm.at[idxs_v.at[b]], bufs_v.at[b], sems[b])

    def wait_gather(b):
        pltpu.make_async_copy(c_hbm.at[idxd_v.at[b]], bufd_v.at[b], semd[b]).wait()
        pltpu.make_async_copy(c_hbm.at[idxs_v.at[b]], bufs_v.at[b], sems[b]).wait()

    def wait_wb(b):
        pltpu.make_async_copy(outb_v.at[b],
                              out_hbm.at[:, pl.ds(0, GCHUNK)], semw[b]).wait()

    def compute(b):
        def row(r, _):
            rcol = jnp.full((LANES,), r, jnp.int32)
            for s in range(HID // LANES):
                sl = pl.ds(s * LANES, LANES)
                sh = pl.ds(HID + s * LANES, LANES)
                v = bufd_v[b, r, sl] + bufs_v[b, r, sh]
                plsc.store_scatter(outb_v.at[b], [lane + s * LANES, rcol], v)
            return 0

        lax.fori_loop(0, GCHUNK, row, 0, unroll=2)

    load_idx_and_gather(0, 0)

    def group(g, _):
        for b in range(2):
            i = g * 2 + b

            @pl.when(i >= 2)
            def _():
                wait_wb(b)

            wait_gather(b)

            @pl.when(i + 1 < _GCH_UNIFORM)
            def _():
                load_idx_and_gather(i + 1, 1 - b)

            compute(b)
            base = (wid + i * NW) * GCHUNK
            pltpu.async_copy(outb_v.at[b],
                             out_hbm.at[:, pl.ds(base, GCHUNK)], semw[b])
        return 0

    lax.fori_loop(0, _GCH_UNIFORM // 2, group, 0)
    wait_wb(0)
    wait_wb(1)

    @pl.when(wid < _GCH_REM)
    def _():
        j = _GCH_UNIFORM * NW + wid
        pltpu.sync_copy(dst_hbm.at[j], idxd_v.at[0])
        pltpu.sync_copy(src_hbm.at[j], idxs_v.at[0])
        pltpu.async_copy(c_hbm.at[idxd_v.at[0]], bufd_v.at[0], semd[0])
        pltpu.async_copy(c_hbm.at[idxs_v.at[0]], bufs_v.at[0], sems[0])
        wait_gather(0)
        compute(0)
        pltpu.sync_copy(outb_v.at[0], out_hbm.at[:, pl.ds(j * GCHUNK, GCHUNK)])


def _edge_gather(c, dst2d, src2d):
    kern = pl.kernel(
        _edge_gather_body,
        out_type=jax.ShapeDtypeStruct((HID, N_EDGES), jnp.float32),
        mesh=_sc_mesh(),
        compiler_params=pltpu.CompilerParams(needs_layout_passes=False),
        scratch_types=[
            pltpu.VMEM((2, GCHUNK), jnp.int32),
            pltpu.VMEM((2, GCHUNK), jnp.int32),
            pltpu.VMEM((2, GCHUNK, 2 * HID), jnp.float32),
            pltpu.VMEM((2, GCHUNK, 2 * HID), jnp.float32),
            pltpu.VMEM((2, HID, GCHUNK), jnp.float32),
            pltpu.SemaphoreType.DMA,
            pltpu.SemaphoreType.DMA,
            pltpu.SemaphoreType.DMA,
            pltpu.SemaphoreType.DMA,
            pltpu.SemaphoreType.DMA,
            pltpu.SemaphoreType.DMA,
        ],
    )
    return kern(c, dst2d, src2d)


_SPILL_CAP = (SCHUNK // LANES) * CPW * LANES


def _seg_max_body(ht_hbm, dst_hbm, out_hbm,
                  acc_v, dstb_v, hb_v, spd_v, spv_v,
                  semd0, semd1, semh0, semh1):
    wid = lax.axis_index("s") * NC + lax.axis_index("c")
    c0 = wid * CPW
    lane = lax.iota(jnp.int32, LANES)
    semd = (semd0, semd1)
    semh = (semh0, semh1)

    def zero(i, _):
        acc_v[pl.ds(i * LANES, LANES)] = jnp.zeros((LANES,), jnp.float32)
        return 0

    lax.fori_loop(0, CPW * N_PAD // LANES, zero, 0)

    def zsp(i, _):
        spd_v[pl.ds(i * LANES, LANES)] = jnp.zeros((LANES,), jnp.int32)
        spv_v[pl.ds(i * LANES, LANES)] = jnp.zeros((LANES,), jnp.float32)
        return 0

    lax.fori_loop(0, _SPILL_CAP // LANES, zsp, 0)

    def issue_loads(i, b):
        base = i * SCHUNK
        pltpu.async_copy(dst_hbm.at[pl.ds(base, SCHUNK)], dstb_v.at[b], semd[b])
        pltpu.async_copy(ht_hbm.at[pl.ds(c0, CPW), pl.ds(base, SCHUNK)],
                         hb_v.at[b], semh[b])

    def wait_loads(b):
        pltpu.make_async_copy(dst_hbm.at[pl.ds(0, SCHUNK)],
                              dstb_v.at[b], semd[b]).wait()
        pltpu.make_async_copy(ht_hbm.at[pl.ds(c0, CPW), pl.ds(0, SCHUNK)],
                              hb_v.at[b], semh[b]).wait()

    def scan_chunk(b):
        def vec(v, off):
            dv = dstb_v[b, pl.ds(v * LANES, LANES)]
            for c in range(CPW):
                idx = dv + c * N_PAD if c else dv
                h = hb_v[b, c, pl.ds(v * LANES, LANES)]
                got = plsc.load_gather(acc_v, [idx])
                m = jnp.maximum(h, got)
                plsc.store_scatter(acc_v, [idx], m)
                got2 = plsc.load_gather(acc_v, [idx])
                lost = m > got2
                pcnt = plsc.all_reduce_population_count(lost)
                plsc.store_scatter(spd_v, [off + lane], idx, mask=lost)
                plsc.store_scatter(spv_v, [off + lane], m, mask=lost)
                off = off + jnp.where(pcnt > 0, LANES, 0)
            return off

        off = lax.fori_loop(0, SCHUNK // LANES, vec,
                            jnp.zeros((LANES,), jnp.int32), unroll=2)
        n = off[0]

        def replay(j, _):
            sdv = spd_v[pl.ds(j * LANES, LANES)]
            sv = spv_v[pl.ds(j * LANES, LANES)]
            got = plsc.load_gather(acc_v, [sdv])
            act = sv > got

            def cond(a):
                return jnp.any(a)

            def body(a):
                plsc.store_scatter(acc_v, [sdv], sv, mask=a)
                g = plsc.load_gather(acc_v, [sdv])
                return a & (sv > g)

            lax.while_loop(cond, body, act)
            return 0

        lax.fori_loop(0, n // LANES, replay, 0)

    issue_loads(0, 0)

    def group(g, _):
        for b in range(2):
            i = g * 2 + b
            wait_loads(b)
            issue_loads(i + 1, 1 - b)
            scan_chunk(b)
        return 0

    lax.fori_loop(0, (NSCHUNKS - 1) // 2, group, 0)
    wait_loads((NSCHUNKS - 1) % 2)
    scan_chunk((NSCHUNKS - 1) % 2)

    for c in range(CPW):
        pltpu.sync_copy(acc_v.at[pl.ds(c * N_PAD, N_PAD)], out_hbm.at[c0 + c])


def _seg_max(ht, dst):
    kern = pl.kernel(
        _seg_max_body,
        out_type=jax.ShapeDtypeStruct((HID, N_PAD), jnp.float32),
        mesh=_sc_mesh(),
        compiler_params=pltpu.CompilerParams(needs_layout_passes=False),
        scratch_types=[
            pltpu.VMEM((CPW * N_PAD,), jnp.float32),
            pltpu.VMEM((2, SCHUNK), jnp.int32),
            pltpu.VMEM((2, CPW, SCHUNK), jnp.float32),
            pltpu.VMEM((_SPILL_CAP,), jnp.int32),
            pltpu.VMEM((_SPILL_CAP,), jnp.float32),
            pltpu.SemaphoreType.DMA,
            pltpu.SemaphoreType.DMA,
            pltpu.SemaphoreType.DMA,
            pltpu.SemaphoreType.DMA,
        ],
    )
    return kern(ht, dst)


def kernel(x, edge_index, W1, b1, W2, b2, W3, b3, W4, b4, Wl, bl):
    src = edge_index[0].astype(jnp.int32)
    dst = edge_index[1].astype(jnp.int32)
    src2d = src.reshape(NCHUNKS, GCHUNK)
    dst2d = dst.reshape(NCHUNKS, GCHUNK)

    w1cat = jnp.concatenate([W1[:IN_CH] - W1[IN_CH:], W1[IN_CH:]], axis=1)
    b1cat = jnp.concatenate([b1, jnp.zeros_like(b1)])[None, :]
    c1 = _node_mm(x, w1cat, b1cat)
    pre1 = _edge_gather(c1, dst2d, src2d)
    h1t = _seg_max(_edge_mm(pre1, W2, b2[:, None]), dst)

    w3cat = jnp.concatenate([W3[:HID] - W3[HID:], W3[HID:]], axis=1)
    b3cat = jnp.concatenate([b3, jnp.zeros_like(b3)])[None, :]
    c2 = _t_mm(h1t, w3cat, b3cat)
    pre2 = _edge_gather(c2, dst2d, src2d)
    h2t = _seg_max(_edge_mm(pre2, W4, b4[:, None]), dst)

    out = _head_mm(h2t, Wl.T, bl[None, :])
    return out[0, :N_NODES]
